# speedup vs baseline: 3.1691x; 3.1691x over previous
import jax
import jax.numpy as jnp
from jax import lax
from jax.experimental import pallas as pl
from jax.experimental.pallas import tpu as pltpu
from jax.experimental.pallas import tpu_sc as plsc

NUM_EMBED = 100000
EMBED_DIM = 128
COL_DIM = 32
BATCH = 16384

NC = 2
NS = 16
NW = NC * NS
B_PER_W = BATCH // NW
CHUNK = 128
NCHUNK = B_PER_W // CHUNK


def _body(table_hbm, idx_hbm, se_hbm, out_hbm, idx_v, rows_v, se_v, sem):
    c = lax.axis_index("c")
    s = lax.axis_index("s")
    wid = s * NC + c

    pltpu.sync_copy(idx_hbm.at[wid], idx_v)
    pltpu.sync_copy(se_hbm, se_v)

    copies = []
    for j in range(NCHUNK):
        copies.append(
            pltpu.async_copy(table_hbm.at[idx_v.at[j]], rows_v.at[j], sem))
    for cp in copies:
        cp.wait()

    s0 = se_v[0]
    s1 = se_v[1]

    def overwrite(r, carry):
        for j in range(NCHUNK):
            rows_v[j, r, pl.ds(0, 16)] = s0
            rows_v[j, r, pl.ds(16, 16)] = s1
        return carry

    lax.fori_loop(0, CHUNK, overwrite, 0)

    pltpu.sync_copy(rows_v, out_hbm.at[wid])


@jax.jit
def _run(idx, table, se2):
    mesh = plsc.VectorSubcoreMesh(core_axis_name="c", subcore_axis_name="s")
    fn = pl.kernel(
        _body,
        mesh=mesh,
        out_type=jax.ShapeDtypeStruct((NW, NCHUNK, CHUNK, EMBED_DIM), jnp.float32),
        scratch_types=[
            pltpu.VMEM((NCHUNK, CHUNK), jnp.int32),
            pltpu.VMEM((NCHUNK, CHUNK, EMBED_DIM), jnp.float32),
            pltpu.VMEM((2, 16), jnp.float32),
            pltpu.SemaphoreType.DMA,
        ],
    )
    return fn(table, idx, se2)


def kernel(X, embed_weight, shared_embed):
    idx = X.astype(jnp.int32).reshape(NW, NCHUNK, CHUNK)
    se2 = shared_embed.reshape(2, 16)
    out = _run(idx, embed_weight, se2)
    return out.reshape(BATCH, EMBED_DIM)
